# Initial kernel scaffold; baseline (speedup 1.0000x reference)
import jax
import jax.numpy as jnp
from jax.experimental import pallas as pl


def kernel(x, alpha):
    raise NotImplementedError("write your pallas kernel here")


# speedup vs baseline: 12.8610x; 12.8610x over previous
import jax
import jax.numpy as jnp
from jax.experimental import pallas as pl
from jax.experimental.pallas import tpu as pltpu

_EPS = 1e-6
_ROWS = 8192
_COLS = 4096
_BLOCK_ROWS = 256


def _quant_body(a_ref, x_ref, out_ref, s_ref):
    a = a_ref[0]
    sp = jnp.maximum(a, 0.0) + jnp.log1p(jnp.exp(-jnp.abs(a)))
    s = jnp.maximum(sp, _EPS)
    s_ref[...] = jnp.full((1, 1), s, jnp.float32)

    x = x_ref[...]
    t = jnp.abs(x) * (32.0 / s)
    u = t + 0.5
    r = jnp.floor(u)
    r = r - jnp.where(r == u, 1.0, 0.0)
    r = jnp.minimum(r, 14.0)
    r = (r
         + jnp.where(t > 15.0, 2.0, 0.0)
         + jnp.where(t > 16.5, 1.0, 0.0)
         + jnp.where(t > 17.5, 1.0, 0.0)
         + jnp.where(t > 19.0, 2.0, 0.0)
         + jnp.where(t > 22.0, 4.0, 0.0))
    out_ref[...] = jnp.sign(x) * (r * (1.0 / 32.0))


def kernel(x, alpha):
    x2 = x.reshape(_ROWS, _COLS)
    a = alpha.reshape(1)
    grid = (_ROWS // _BLOCK_ROWS,)
    out, s = pl.pallas_call(
        _quant_body,
        grid=grid,
        in_specs=[
            pl.BlockSpec(memory_space=pltpu.SMEM),
            pl.BlockSpec((_BLOCK_ROWS, _COLS), lambda i: (i, 0)),
        ],
        out_specs=[
            pl.BlockSpec((_BLOCK_ROWS, _COLS), lambda i: (i, 0)),
            pl.BlockSpec((1, 1), lambda i: (0, 0)),
        ],
        out_shape=[
            jax.ShapeDtypeStruct((_ROWS, _COLS), jnp.float32),
            jax.ShapeDtypeStruct((1, 1), jnp.float32),
        ],
    )(a, x2)
    return out.reshape(x.shape), s.reshape(())
